# baseline (speedup 1.0000x reference)
import jax
import jax.numpy as jnp
from jax.experimental import pallas as pl
from jax.experimental.pallas import tpu as pltpu

N_BARS = 256
B_STATIC = 64
EMB_D = 1024
N_SEM = 8


def _body(idx_ref, emb_ref, out_ref, gath_ref, sems):
    idx = idx_ref[...]
    cols = jax.lax.broadcasted_iota(jnp.int32, (N_BARS, N_BARS), 1)
    onehot = (idx == cols).astype(jnp.float32)
    gath_ref[...] = jnp.dot(onehot, emb_ref[...],
                            preferred_element_type=jnp.float32)
    half = N_BARS // 2
    copies = [
        pltpu.make_async_copy(gath_ref.at[pl.ds(h * half, half)],
                              out_ref.at[j, pl.ds(h * half, half)],
                              sems.at[(2 * j + h) % N_SEM])
        for j in range(B_STATIC) for h in range(2)
    ]
    for c in copies:
        c.start()
    for c in copies:
        c.wait()


def kernel(num_bars, batch_size, embedding):
    shift = (num_bars - N_BARS) + (batch_size - B_STATIC)
    idx = jnp.clip(jnp.arange(N_BARS, dtype=jnp.int32) + shift, 0, N_BARS - 1)
    idx2 = idx.reshape(N_BARS, 1)

    out = pl.pallas_call(
        _body,
        in_specs=[
            pl.BlockSpec(memory_space=pltpu.VMEM),
            pl.BlockSpec(memory_space=pltpu.VMEM),
        ],
        out_specs=pl.BlockSpec(memory_space=pl.ANY),
        out_shape=jax.ShapeDtypeStruct((B_STATIC, N_BARS, EMB_D), jnp.float32),
        scratch_shapes=[
            pltpu.VMEM((N_BARS, EMB_D), jnp.float32),
            pltpu.SemaphoreType.DMA((N_SEM,)),
        ],
    )(idx2, embedding)
    return out


# speedup vs baseline: 1.0361x; 1.0361x over previous
import jax
import jax.numpy as jnp
from jax.experimental import pallas as pl
from jax.experimental.pallas import tpu as pltpu

N_BARS = 256
B_STATIC = 64
EMB_D = 1024
N_SEM = 8


def _body(shift_ref, emb_ref, out_ref, gath_ref, sems):
    shift = shift_ref[0]
    emb = emb_ref[...]
    rolled = pltpu.roll(emb, -shift, 0)
    pos = jax.lax.broadcasted_iota(jnp.int32, (N_BARS, EMB_D), 0) + shift
    oob = (pos >= N_BARS) | (pos < -N_BARS)
    gath_ref[...] = jnp.where(oob, jnp.nan, rolled)
    copies = [
        pltpu.make_async_copy(gath_ref, out_ref.at[j], sems.at[j % N_SEM])
        for j in range(B_STATIC)
    ]
    for c in copies:
        c.start()
    for c in copies:
        c.wait()


def kernel(num_bars, batch_size, embedding):
    shift = (num_bars - N_BARS) + (batch_size - B_STATIC)
    shift_arr = jnp.asarray(shift, jnp.int32).reshape(1)

    out = pl.pallas_call(
        _body,
        in_specs=[
            pl.BlockSpec(memory_space=pltpu.SMEM),
            pl.BlockSpec(memory_space=pltpu.VMEM),
        ],
        out_specs=pl.BlockSpec(memory_space=pl.ANY),
        out_shape=jax.ShapeDtypeStruct((B_STATIC, N_BARS, EMB_D), jnp.float32),
        scratch_shapes=[
            pltpu.VMEM((N_BARS, EMB_D), jnp.float32),
            pltpu.SemaphoreType.DMA((N_SEM,)),
        ],
    )(shift_arr, embedding)
    return out
